# baseline (speedup 1.0000x reference)
import jax
import jax.numpy as jnp
import numpy as np
from jax import lax
from jax.experimental import pallas as pl
from jax.experimental.pallas import tpu as pltpu
from jax.experimental.pallas import tpu_sc as plsc

N_IN = 10000
D = 256
E = 513
E_PAD = 520
N_OUT = N_IN + E
NC, NS = 2, 16
NW = NC * NS
EPW = 16
EPC = 24
LANES = 16

ROWS_A, ROWS_B = 320, 312
SPLIT_W = 2
SPLIT_ROW = SPLIT_W * ROWS_A
CHUNK = 64


def _pool_kernel(x_hbm, i0_hbm, i1_hbm, orow_hbm, out_hbm,
                 idx0_v, idx1_v, orow_v, buf0, buf1, sem0, sem1, semc):
    c = lax.axis_index("c")
    s = lax.axis_index("s")
    wid = s * NC + c

    def run(rows, base):
        hcopy = pltpu.async_copy(x_hbm.at[pl.ds(base, rows)],
                                 out_hbm.at[pl.ds(base, rows)], semc)
        ebase = wid * EPW
        pltpu.sync_copy(i0_hbm.at[pl.ds(ebase, EPC)], idx0_v)
        pltpu.sync_copy(i1_hbm.at[pl.ds(ebase, EPC)], idx1_v)
        pltpu.sync_copy(orow_hbm.at[pl.ds(ebase, EPC)], orow_v)
        hg0 = pltpu.async_copy(x_hbm.at[idx0_v], buf0, sem0)
        hg1 = pltpu.async_copy(x_hbm.at[idx1_v], buf1, sem1)
        hg0.wait()
        hg1.wait()

        def body(e, carry):
            for j in range(D // LANES):
                sl = pl.ds(j * LANES, LANES)
                buf0[e, sl] = (buf0[e, sl] + buf1[e, sl]) * 0.5
            return carry

        lax.fori_loop(0, EPC, body, 0)

        pltpu.sync_copy(buf0, out_hbm.at[orow_v])

        hcopy.wait()

    @pl.when(wid < SPLIT_W)
    def _():
        run(ROWS_A, wid * ROWS_A)

    @pl.when(wid >= SPLIT_W)
    def _():
        run(ROWS_B, SPLIT_ROW + (wid - SPLIT_W) * ROWS_B)


_OROW_NP = (np.minimum(np.arange(E_PAD), E - 1) + N_IN).astype(np.int32)


@jax.jit
def _run(x, idx0, idx1):
    orow = jnp.asarray(_OROW_NP)
    mesh = plsc.VectorSubcoreMesh(core_axis_name="c", subcore_axis_name="s",
                                  num_cores=NC, num_subcores=NS)
    return pl.kernel(
        _pool_kernel,
        out_type=jax.ShapeDtypeStruct((N_OUT, D), jnp.float32),
        mesh=mesh,
        scratch_types=[
            pltpu.VMEM((EPC,), jnp.int32),
            pltpu.VMEM((EPC,), jnp.int32),
            pltpu.VMEM((EPC,), jnp.int32),
            pltpu.VMEM((EPC, D), jnp.float32),
            pltpu.VMEM((EPC, D), jnp.float32),
            pltpu.SemaphoreType.DMA,
            pltpu.SemaphoreType.DMA,
            pltpu.SemaphoreType.DMA,
        ],
    )(x, idx0, idx1, orow)


def kernel(input, pool_idx):
    idx = pool_idx.astype(jnp.int32)
    pad = jnp.broadcast_to(idx[-1:], (E_PAD - E, 2))
    idx = jnp.concatenate([idx, pad], axis=0)
    return _run(input, idx[:, 0], idx[:, 1])


# speedup vs baseline: 11.0367x; 11.0367x over previous
import jax
import jax.numpy as jnp
import numpy as np
from jax import lax
from jax.experimental import pallas as pl
from jax.experimental.pallas import tpu as pltpu
from jax.experimental.pallas import tpu_sc as plsc

N_IN = 10000
D = 256
E = 513
E_PAD = 520
N_OUT = N_IN + E
NC, NS = 2, 16
NW = NC * NS
EPW = 16
EPC = 24
LANES = 16

ROWS_A, ROWS_B = 320, 312
SPLIT_W = 2
SPLIT_ROW = SPLIT_W * ROWS_A
CHUNK = 64


def _pool_kernel(x_hbm, i0_hbm, i1_hbm, orow_hbm, out_hbm,
                 idx0_v, idx1_v, orow_v, buf0, buf1, sem0, sem1,
                 stages, sems_in, sems_out):
    c = lax.axis_index("c")
    s = lax.axis_index("s")
    wid = s * NC + c

    def run(sizes, base):
        offs, o = [], 0
        for sz in sizes:
            offs.append(o)
            o += sz
        hin = [pltpu.async_copy(x_hbm.at[pl.ds(base + off, sz)],
                                stages[i].at[pl.ds(0, sz)], sems_in[i])
               for i, (off, sz) in enumerate(zip(offs, sizes))]
        ebase = wid * EPW
        pltpu.sync_copy(i0_hbm.at[pl.ds(ebase, EPC)], idx0_v)
        pltpu.sync_copy(i1_hbm.at[pl.ds(ebase, EPC)], idx1_v)
        pltpu.sync_copy(orow_hbm.at[pl.ds(ebase, EPC)], orow_v)
        hg0 = pltpu.async_copy(x_hbm.at[idx0_v], buf0, sem0)
        hg1 = pltpu.async_copy(x_hbm.at[idx1_v], buf1, sem1)
        hout = []
        for i, (off, sz) in enumerate(zip(offs, sizes)):
            hin[i].wait()
            hout.append(pltpu.async_copy(stages[i].at[pl.ds(0, sz)],
                                         out_hbm.at[pl.ds(base + off, sz)],
                                         sems_out[i]))
        hg0.wait()
        hg1.wait()

        def body(e, carry):
            for j in range(D // LANES):
                sl = pl.ds(j * LANES, LANES)
                buf0[e, sl] = (buf0[e, sl] + buf1[e, sl]) * 0.5
            return carry

        lax.fori_loop(0, EPC, body, 0)

        pltpu.sync_copy(buf0, out_hbm.at[orow_v])

        for h in hout:
            h.wait()

    @pl.when(wid < SPLIT_W)
    def _():
        run((CHUNK,) * 5, wid * ROWS_A)

    @pl.when(wid >= SPLIT_W)
    def _():
        run((CHUNK,) * 4 + (ROWS_B - 4 * CHUNK,),
            SPLIT_ROW + (wid - SPLIT_W) * ROWS_B)


_OROW_NP = (np.minimum(np.arange(E_PAD), E - 1) + N_IN).astype(np.int32)


@jax.jit
def _run(x, idx0, idx1):
    orow = jnp.asarray(_OROW_NP)
    mesh = plsc.VectorSubcoreMesh(core_axis_name="c", subcore_axis_name="s",
                                  num_cores=NC, num_subcores=NS)
    return pl.kernel(
        _pool_kernel,
        out_type=jax.ShapeDtypeStruct((N_OUT, D), jnp.float32),
        mesh=mesh,
        scratch_types=[
            pltpu.VMEM((EPC,), jnp.int32),
            pltpu.VMEM((EPC,), jnp.int32),
            pltpu.VMEM((EPC,), jnp.int32),
            pltpu.VMEM((EPC, D), jnp.float32),
            pltpu.VMEM((EPC, D), jnp.float32),
            pltpu.SemaphoreType.DMA,
            pltpu.SemaphoreType.DMA,
            [pltpu.VMEM((CHUNK, D), jnp.float32) for _ in range(5)],
            [pltpu.SemaphoreType.DMA for _ in range(5)],
            [pltpu.SemaphoreType.DMA for _ in range(5)],
        ],
    )(x, idx0, idx1, orow)


def kernel(input, pool_idx):
    idx = pool_idx.astype(jnp.int32)
    pad = jnp.broadcast_to(idx[-1:], (E_PAD - E, 2))
    idx = jnp.concatenate([idx, pad], axis=0)
    return _run(input, idx[:, 0], idx[:, 1])
